# Initial kernel scaffold; baseline (speedup 1.0000x reference)
import jax
import jax.numpy as jnp
from jax.experimental import pallas as pl


def kernel(similarities, category_ids, original_sizes):
    raise NotImplementedError("write your pallas kernel here")


# speedup vs baseline: 111.0300x; 111.0300x over previous
import jax
import jax.numpy as jnp
from jax.experimental import pallas as pl
from jax.experimental.pallas import tpu as pltpu

_T, _C, _H, _W = 4, 16, 512, 512
_G = 16
_CS = 32
_THRESH = 0.65
_NFG = 40
_MAXP = 42
_OUTROWS = 48
_NEG = -1e30
def _map_body(sizes_ref, sim_ref, out_ref):
    _BIGI = jnp.int32(1 << 28)
    t = pl.program_id(0)
    x = sim_ref[0, 0]
    m = jnp.where(x > _THRESH, x, jnp.float32(-jnp.inf))

    col_iota = jax.lax.broadcasted_iota(jnp.int32, (_H, _CS), 1)

    amax, aarg, rmin_l, rarg_l = [], [], [], []
    for j in range(_G):
        xj = x[:, j * _CS:(j + 1) * _CS]
        mj = m[:, j * _CS:(j + 1) * _CS]
        cj = col_iota + (j * _CS)
        mx = jnp.max(mj, axis=1, keepdims=True)
        amax.append(mx)
        aarg.append(jnp.min(jnp.where(mj == mx, cj, _BIGI), axis=1,
                            keepdims=True))
        mn = jnp.min(xj, axis=1, keepdims=True)
        rmin_l.append(mn)
        rarg_l.append(jnp.min(jnp.where(xj == mn, cj, _BIGI), axis=1,
                              keepdims=True))
    a_max = jnp.concatenate(amax, axis=1)
    a_arg = jnp.concatenate(aarg, axis=1)
    r_min = jnp.concatenate(rmin_l, axis=1)
    r_arg = jnp.concatenate(rarg_l, axis=1)

    row_iota32 = jax.lax.broadcasted_iota(jnp.int32, (_CS, _G), 0)
    bmax, barg = [], []
    for i in range(_G):
        ai = a_max[i * _CS:(i + 1) * _CS, :]
        gi = a_arg[i * _CS:(i + 1) * _CS, :]
        mx = jnp.max(ai, axis=0, keepdims=True)
        flat = (row_iota32 + i * _CS) * _W + gi
        barg.append(jnp.min(jnp.where(ai == mx, flat, _BIGI), axis=0,
                            keepdims=True))
        bmax.append(mx)
    cell_max = jnp.concatenate(bmax, axis=0)
    cell_arg = jnp.concatenate(barg, axis=0)

    row_min = jnp.min(r_min, axis=1, keepdims=True)
    row_col = jnp.min(jnp.where(r_min == row_min, r_arg, _BIGI), axis=1,
                      keepdims=True)
    g_min = jnp.min(row_min)
    row_iota512 = jax.lax.broadcasted_iota(jnp.int32, (_H, 1), 0)
    g_flat = jnp.min(jnp.where(row_min == g_min, row_iota512 * _W + row_col,
                               _BIGI))

    valid = cell_max > _THRESH
    n_fg = jnp.minimum(jnp.sum(valid.astype(jnp.int32)), _NFG)
    cell_id = (jax.lax.broadcasted_iota(jnp.int32, (_G, _G), 0) * _G
               + jax.lax.broadcasted_iota(jnp.int32, (_G, _G), 1))
    skey = jnp.where(valid, cell_max, jnp.float32(_NEG))

    ori_h = sizes_ref[t, 0].astype(jnp.float32)
    ori_w = sizes_ref[t, 1].astype(jnp.float32)
    scale_x = ori_w / _W
    scale_y = ori_h / _H

    out_r = jax.lax.broadcasted_iota(jnp.int32, (_OUTROWS, 4), 0)
    out_c = jax.lax.broadcasted_iota(jnp.int32, (_OUTROWS, 4), 1)
    acc = jnp.zeros((_OUTROWS, 4), jnp.float32)
    for k in range(_NFG):
        mk = jnp.max(skey)
        sel = jnp.min(jnp.where(skey == mk, cell_id, jnp.int32(1 << 12)))
        fidx = jnp.min(jnp.where(cell_id == sel, cell_arg, _BIGI))
        vk = (mk > _THRESH).astype(jnp.float32)
        fx = (fidx % _W).astype(jnp.float32)
        fy = (fidx // _W).astype(jnp.float32)
        r0 = fx * scale_x * vk
        r1 = fy * scale_y * vk
        r2 = mk * vk
        r3 = vk
        vals = jnp.where(out_c == 0, r0,
                         jnp.where(out_c == 1, r1,
                                   jnp.where(out_c == 2, r2, r3)))
        acc = jnp.where(out_r == k, vals, acc)
        skey = jnp.where(cell_id == sel, jnp.float32(_NEG), skey)

    bx = (g_flat % _W).astype(jnp.float32) * scale_x
    by = (g_flat // _W).astype(jnp.float32) * scale_y
    bvals = jnp.where(out_c == 0, bx,
                      jnp.where(out_c == 1, by,
                                jnp.where(out_c == 2, g_min,
                                          jnp.float32(0.0))))
    acc = jnp.where(out_r == n_fg, bvals, acc)
    n_pts = (n_fg + 1).astype(jnp.float32)
    acc = jnp.where((out_r == _MAXP) & (out_c == 0), n_pts, acc)

    out_ref[0, 0] = acc


def kernel(similarities, category_ids, original_sizes):
    del category_ids
    grid_spec = pltpu.PrefetchScalarGridSpec(
        num_scalar_prefetch=1,
        grid=(_T, _C),
        in_specs=[
            pl.BlockSpec((1, 1, _H, _W), lambda t, c, sizes: (t, c, 0, 0)),
        ],
        out_specs=pl.BlockSpec((1, 1, _OUTROWS, 4),
                               lambda t, c, sizes: (t, c, 0, 0)),
    )
    raw = pl.pallas_call(
        _map_body,
        grid_spec=grid_spec,
        out_shape=jax.ShapeDtypeStruct((_T, _C, _OUTROWS, 4), jnp.float32),
        compiler_params=pltpu.CompilerParams(
            dimension_semantics=("parallel", "parallel"),
        ),
    )(original_sizes, similarities)
    point_prompts = raw[:, :, :_MAXP, :]
    num_points = raw[:, :, _MAXP, 0].astype(jnp.int32)
    return point_prompts, num_points


# speedup vs baseline: 1764.8459x; 15.8952x over previous
import jax
import jax.numpy as jnp
from jax.experimental import pallas as pl
from jax.experimental.pallas import tpu as pltpu

_T, _C, _H, _W = 4, 16, 512, 512
_G = 16
_CS = 32
_THRESH = 0.65
_NFG = 40
_MAXP = 42
_NMAP = _T * _C
_NEG = -1e30
_PACK = 640


def _reduce_body(sim_ref, out_ref):
    _BIGI = jnp.int32(1 << 28)
    x = sim_ref[0, 0]
    m = jnp.where(x > _THRESH, x, jnp.float32(-jnp.inf))

    smax_l, srow_l, smin_l, srmin_l = [], [], [], []
    riota = jax.lax.broadcasted_iota(jnp.int32, (_CS, _W), 0)
    for i in range(_G):
        xi = x[i * _CS:(i + 1) * _CS, :]
        mi = m[i * _CS:(i + 1) * _CS, :]
        ri = riota + (i * _CS)
        mx = jnp.max(mi, axis=0, keepdims=True)
        smax_l.append(mx)
        srow_l.append(jnp.min(jnp.where(mi == mx, ri, _BIGI), axis=0,
                              keepdims=True))
        mn = jnp.min(xi, axis=0, keepdims=True)
        smin_l.append(mn)
        srmin_l.append(jnp.min(jnp.where(xi == mn, ri, _BIGI), axis=0,
                               keepdims=True))
    smax = jnp.concatenate(smax_l, axis=0)
    srow = jnp.concatenate(srow_l, axis=0)
    smin = jnp.concatenate(smin_l, axis=0)
    srmin = jnp.concatenate(srmin_l, axis=0)

    ciota = jax.lax.broadcasted_iota(jnp.int32, (_G, _CS), 1)
    cmax_l, carg_l = [], []
    for j in range(_G):
        sj = smax[:, j * _CS:(j + 1) * _CS]
        rj = srow[:, j * _CS:(j + 1) * _CS]
        cj = ciota + (j * _CS)
        mx = jnp.max(sj, axis=1, keepdims=True)
        cmax_l.append(mx)
        flat = rj * _W + cj
        carg_l.append(jnp.min(jnp.where(sj == mx, flat, _BIGI), axis=1,
                              keepdims=True))
    cell_max = jnp.concatenate(cmax_l, axis=1)
    cell_arg = jnp.concatenate(carg_l, axis=1)

    g_min = jnp.min(smin)
    colabs = jax.lax.broadcasted_iota(jnp.int32, (_G, _W), 1)
    g_flat = jnp.min(jnp.where(smin == g_min, srmin * _W + colabs, _BIGI))

    srow_out = jnp.concatenate([cell_max[i:i + 1, :] for i in range(_G)],
                               axis=1)
    arow_out = jnp.concatenate(
        [cell_arg[i:i + 1, :].astype(jnp.float32) for i in range(_G)],
        axis=1)
    li = jax.lax.broadcasted_iota(jnp.int32, (1, 128), 1)
    bvec = jnp.where(li == 0, g_flat.astype(jnp.float32),
                     jnp.where(li == 1, g_min, jnp.float32(0.0)))
    out_ref[0] = jnp.concatenate([srow_out, arow_out, bvec], axis=1)


def _select_body(sizes_ref, packed_ref, out_ref):
    X = packed_ref[:, :]
    scores = X[:, 0:256]
    argsf = X[:, 256:512]
    gflatf = X[:, 512:513]
    gminv = X[:, 513:514]

    valid = scores > _THRESH
    nfg = jnp.minimum(jnp.sum(valid.astype(jnp.int32), axis=1,
                              keepdims=True), _NFG)
    skey = jnp.where(valid, scores, jnp.float32(_NEG))
    cid = jax.lax.broadcasted_iota(jnp.int32, (_NMAP, 256), 1)

    fx_l, fy_l, fs_l, fv_l = [], [], [], []
    for _ in range(_NFG):
        mk = jnp.max(skey, axis=1, keepdims=True)
        sel = jnp.min(jnp.where(skey == mk, cid, jnp.int32(4096)), axis=1,
                      keepdims=True)
        hit = cid == sel
        fidx = jnp.min(jnp.where(hit, argsf, jnp.float32(1 << 28)), axis=1,
                       keepdims=True)
        vk = (mk > _THRESH).astype(jnp.float32)
        fy = jnp.floor(fidx * (1.0 / _W))
        fx = fidx - fy * _W
        fx_l.append(fx)
        fy_l.append(fy)
        fs_l.append(mk)
        fv_l.append(vk)
        skey = jnp.where(hit, jnp.float32(_NEG), skey)

    zpad = [jnp.zeros((_NMAP, 1), jnp.float32)] * 8
    planeX = jnp.concatenate(fx_l + zpad, axis=1)
    planeY = jnp.concatenate(fy_l + zpad, axis=1)
    planeS = jnp.concatenate(fs_l + zpad, axis=1)
    planeV = jnp.concatenate(fv_l + zpad, axis=1)

    tcol = jax.lax.broadcasted_iota(jnp.int32, (_NMAP, 1), 0) // _C
    oriw = jnp.zeros((_NMAP, 1), jnp.float32)
    orih = jnp.zeros((_NMAP, 1), jnp.float32)
    for t in range(_T):
        oriw = jnp.where(tcol == t, sizes_ref[t, 1].astype(jnp.float32),
                         oriw)
        orih = jnp.where(tcol == t, sizes_ref[t, 0].astype(jnp.float32),
                         orih)
    scale_x = oriw * (1.0 / _W)
    scale_y = orih * (1.0 / _H)

    planeX = planeX * scale_x * planeV
    planeY = planeY * scale_y * planeV
    planeS = planeS * planeV

    lane48 = jax.lax.broadcasted_iota(jnp.int32, (_NMAP, 48), 1)
    bmask = lane48 == nfg
    gy = jnp.floor(gflatf * (1.0 / _W))
    gx = gflatf - gy * _W
    planeX = jnp.where(bmask, gx * scale_x, planeX)
    planeY = jnp.where(bmask, gy * scale_y, planeY)
    planeS = jnp.where(bmask, gminv, planeS)
    planeV = jnp.where(bmask, jnp.float32(0.0), planeV)
    planeN = (nfg.astype(jnp.float32) + 1.0) + jnp.zeros((_NMAP, 48),
                                                         jnp.float32)

    out_ref[0] = planeX
    out_ref[1] = planeY
    out_ref[2] = planeS
    out_ref[3] = planeV
    out_ref[4] = planeN


def kernel(similarities, category_ids, original_sizes):
    del category_ids
    packed = pl.pallas_call(
        _reduce_body,
        grid=(_NMAP,),
        in_specs=[pl.BlockSpec((1, 1, _H, _W),
                               lambda i: (i // _C, i % _C, 0, 0))],
        out_specs=pl.BlockSpec((1, 1, _PACK), lambda i: (i, 0, 0)),
        out_shape=jax.ShapeDtypeStruct((_NMAP, 1, _PACK), jnp.float32),
        compiler_params=pltpu.CompilerParams(
            dimension_semantics=("parallel",),
        ),
    )(similarities)
    packed2 = packed.reshape(_NMAP, _PACK)

    grid_spec = pltpu.PrefetchScalarGridSpec(
        num_scalar_prefetch=1,
        grid=(1,),
        in_specs=[pl.BlockSpec((_NMAP, _PACK), lambda i, s: (0, 0))],
        out_specs=pl.BlockSpec((5, _NMAP, 48), lambda i, s: (0, 0, 0)),
    )
    raw = pl.pallas_call(
        _select_body,
        grid_spec=grid_spec,
        out_shape=jax.ShapeDtypeStruct((5, _NMAP, 48), jnp.float32),
    )(original_sizes, packed2)

    pp = jnp.transpose(raw[:4], (1, 2, 0))
    point_prompts = pp[:, :_MAXP, :].reshape(_T, _C, _MAXP, 4)
    num_points = raw[4, :, 0].astype(jnp.int32).reshape(_T, _C)
    return point_prompts, num_points
